# Initial kernel scaffold; baseline (speedup 1.0000x reference)
import jax
import jax.numpy as jnp
from jax.experimental import pallas as pl


def kernel(input_variable, weight):
    raise NotImplementedError("write your pallas kernel here")


# speedup vs baseline: 1.0234x; 1.0234x over previous
import functools

import jax
import jax.numpy as jnp
from jax import lax
from jax.experimental import pallas as pl
from jax.experimental.pallas import tpu as pltpu
from jax.experimental.pallas import tpu_sc as plsc

_NC = 2
_NS = 16
_NW = _NC * _NS
_CHUNK = 128


@functools.partial(jax.jit, static_argnames=("n_chunks", "dim"))
def _sc_gather(weight, idx3, n_chunks, dim):
    b_per_w = n_chunks * _CHUNK
    n_rows = _NW * b_per_w
    mesh = plsc.VectorSubcoreMesh(core_axis_name="c", subcore_axis_name="s")

    def body(w_hbm, idx_hbm, out_hbm, idx_v, row_v, sem):
        wid = lax.axis_index("s") * _NC + lax.axis_index("c")
        base = wid * b_per_w
        pltpu.sync_copy(idx_hbm.at[wid], idx_v)

        def chunk(j, carry):
            pltpu.async_copy(w_hbm.at[idx_v.at[j]], row_v, sem).wait()
            pltpu.sync_copy(row_v, out_hbm.at[pl.ds(base + j * _CHUNK, _CHUNK)])
            return carry

        lax.fori_loop(0, n_chunks, chunk, 0)

    fn = pl.kernel(
        body,
        out_type=jax.ShapeDtypeStruct((n_rows, dim), jnp.float32),
        mesh=mesh,
        scratch_types=[
            pltpu.VMEM((n_chunks, _CHUNK), jnp.int32),
            pltpu.VMEM((_CHUNK, dim), jnp.float32),
            pltpu.SemaphoreType.DMA,
        ],
        compiler_params=pltpu.CompilerParams(use_tc_tiling_on_sc=False),
    )
    return fn(weight, idx3)


def kernel(input_variable, weight):
    dim = weight.shape[1]
    flat = input_variable.reshape(-1).astype(jnp.int32)
    n = flat.shape[0]
    per_w = -(-n // (_NW * _CHUNK)) * _CHUNK
    n_pad = _NW * per_w
    if n_pad != n:
        flat = jnp.pad(flat, (0, n_pad - n))
    idx3 = flat.reshape(_NW, per_w // _CHUNK, _CHUNK)
    out = _sc_gather(weight, idx3, per_w // _CHUNK, dim)
    if n_pad != n:
        out = out[:n]
    return out.reshape(*input_variable.shape, dim)


# speedup vs baseline: 1.1057x; 1.0804x over previous
import functools

import jax
import jax.numpy as jnp
from jax import lax
from jax.experimental import pallas as pl
from jax.experimental.pallas import tpu as pltpu
from jax.experimental.pallas import tpu_sc as plsc

_NC = 2
_NS = 16
_NW = _NC * _NS
_CHUNK = 128
_NBUF = 10


@functools.partial(jax.jit, static_argnames=("n_chunks", "dim"))
def _sc_gather(weight, idx3, n_chunks, dim):
    b_per_w = n_chunks * _CHUNK
    n_rows = _NW * b_per_w
    mesh = plsc.VectorSubcoreMesh(core_axis_name="c", subcore_axis_name="s")

    nb = _NBUF

    def body(w_hbm, idx_hbm, out_hbm, idx_v, rows, gsem, wsem):
        wid = lax.axis_index("s") * _NC + lax.axis_index("c")
        base = wid * b_per_w
        pltpu.sync_copy(idx_hbm.at[wid], idx_v)

        def group(g, carry):
            j0 = g * nb
            for b in range(nb):
                pltpu.async_copy(w_hbm.at[idx_v.at[j0 + b]], rows.at[b], gsem)
            for b in range(nb):
                pltpu.make_async_copy(
                    w_hbm.at[idx_v.at[j0 + b]], rows.at[b], gsem
                ).wait()
            for b in range(nb):
                pltpu.async_copy(
                    rows.at[b],
                    out_hbm.at[pl.ds(base + (j0 + b) * _CHUNK, _CHUNK)],
                    wsem,
                )
            for b in range(nb):
                pltpu.make_async_copy(
                    rows.at[b],
                    out_hbm.at[pl.ds(base + (j0 + b) * _CHUNK, _CHUNK)],
                    wsem,
                ).wait()
            return carry

        lax.fori_loop(0, n_chunks // nb, group, 0)

    fn = pl.kernel(
        body,
        out_type=jax.ShapeDtypeStruct((n_rows, dim), jnp.float32),
        mesh=mesh,
        scratch_types=[
            pltpu.VMEM((n_chunks, _CHUNK), jnp.int32),
            pltpu.VMEM((nb, _CHUNK, dim), jnp.float32),
            pltpu.SemaphoreType.DMA,
            pltpu.SemaphoreType.DMA,
        ],
        compiler_params=pltpu.CompilerParams(use_tc_tiling_on_sc=False),
    )
    return fn(weight, idx3)


def kernel(input_variable, weight):
    dim = weight.shape[1]
    flat = input_variable.reshape(-1).astype(jnp.int32)
    n = flat.shape[0]
    grain = _CHUNK * _NBUF
    per_w = -(-n // (_NW * grain)) * grain
    n_pad = _NW * per_w
    if n_pad != n:
        flat = jnp.pad(flat, (0, n_pad - n))
    idx3 = flat.reshape(_NW, per_w // _CHUNK, _CHUNK)
    out = _sc_gather(weight, idx3, per_w // _CHUNK, dim)
    if n_pad != n:
        out = out[:n]
    return out.reshape(*input_variable.shape, dim)


# speedup vs baseline: 1.7677x; 1.5988x over previous
import functools

import jax
import jax.numpy as jnp
from jax import lax
from jax.experimental import pallas as pl
from jax.experimental.pallas import tpu as pltpu
from jax.experimental.pallas import tpu_sc as plsc

_NC = 2
_NS = 16
_NW = _NC * _NS
_WG = 16


@functools.partial(jax.jit, static_argnames=("rows_per_w", "hist", "dim"))
def _sc_embed(weight, idx, rows_per_w, hist, dim):
    batch = rows_per_w * _NW
    n_groups = rows_per_w // _WG
    mesh = plsc.VectorSubcoreMesh(core_axis_name="c", subcore_axis_name="s")

    def body(w_hbm, idx_hbm, out_hbm, idx_v, wide, gsem, wsem):
        wid = lax.axis_index("s") * _NC + lax.axis_index("c")
        base = wid * rows_per_w
        pltpu.sync_copy(idx_hbm.at[pl.ds(base, rows_per_w)], idx_v)

        def group(g, carry):
            r0 = g * _WG
            for k in range(_WG):
                pltpu.async_copy(w_hbm.at[idx_v.at[r0 + k]], wide.at[k], gsem)
            for k in range(_WG):
                pltpu.make_async_copy(
                    w_hbm.at[idx_v.at[r0 + k]], wide.at[k], gsem
                ).wait()
            copy = pltpu.make_async_copy(
                wide, out_hbm.at[pl.ds(base + r0, _WG)], wsem
            )
            copy.start()
            copy.wait()
            return carry

        lax.fori_loop(0, n_groups, group, 0)

    fn = pl.kernel(
        body,
        out_type=jax.ShapeDtypeStruct((batch, hist, dim), jnp.float32),
        mesh=mesh,
        scratch_types=[
            pltpu.VMEM((rows_per_w, hist), jnp.int32),
            pltpu.VMEM((_WG, hist, dim), jnp.float32),
            pltpu.SemaphoreType.DMA,
            pltpu.SemaphoreType.DMA,
        ],
        compiler_params=pltpu.CompilerParams(use_tc_tiling_on_sc=False),
    )
    return fn(weight, idx)


def kernel(input_variable, weight):
    dim = weight.shape[1]
    batch, hist = input_variable.shape
    idx = input_variable.astype(jnp.int32)
    grain = _NW * _WG
    batch_pad = -(-batch // grain) * grain
    if batch_pad != batch:
        idx = jnp.pad(idx, ((0, batch_pad - batch), (0, 0)))
    out = _sc_embed(weight, idx, batch_pad // _NW, hist, dim)
    if batch_pad != batch:
        out = out[:batch]
    return out
